# Initial kernel scaffold; baseline (speedup 1.0000x reference)
import jax
import jax.numpy as jnp
from jax.experimental import pallas as pl


def kernel(embeddings, labels):
    raise NotImplementedError("write your pallas kernel here")


# speedup vs baseline: 17.1703x; 17.1703x over previous
import jax
import jax.numpy as jnp
from jax.experimental import pallas as pl
from jax.experimental.pallas import tpu as pltpu

_C = 16
_N = 4096
_D = 512
_ALPHA = 0.5
_BETA = 0.5


def _island_body(e_ref, l_ref, o_ref):
    e = e_ref[...]
    lab = l_ref[...]
    classes = jax.lax.broadcasted_iota(jnp.int32, (_N, _C), 1)
    onehot = (lab == classes).astype(jnp.float32)
    dn = (((0,), (0,)), ((), ()))
    sums = jax.lax.dot_general(onehot, e, dn,
                               preferred_element_type=jnp.float32)
    sqs = jax.lax.dot_general(onehot, e * e, dn,
                              preferred_element_type=jnp.float32)
    counts = jnp.sum(onehot, axis=0, keepdims=True)
    safe = jnp.maximum(counts, 1.0)
    s2 = jnp.sum(sqs, axis=1, keepdims=True)
    p2 = jnp.sum(sums * sums, axis=1, keepdims=True)
    intra_c = (s2 - p2 / safe.T) / (safe.T * _D)
    intra = jnp.sum(jnp.where(counts.T > 1.0, intra_c, 0.0))
    means = sums / safe.T
    mnorm2 = jnp.sum(means * means)
    tot = jnp.sum(means, axis=0, keepdims=True)
    inter = (_C * mnorm2 - jnp.sum(tot * tot)) / _D
    o_ref[0, 0] = _ALPHA * intra + _BETA * inter


def kernel(embeddings, labels):
    lab2d = jnp.asarray(labels, jnp.int32).reshape(_N, 1)
    out = pl.pallas_call(
        _island_body,
        out_shape=jax.ShapeDtypeStruct((1, 1), jnp.float32),
        in_specs=[
            pl.BlockSpec(memory_space=pltpu.VMEM),
            pl.BlockSpec(memory_space=pltpu.VMEM),
        ],
        out_specs=pl.BlockSpec(memory_space=pltpu.SMEM),
    )(embeddings, lab2d)
    return out[0, 0]
